# baseline (speedup 1.0000x reference)
from functools import partial

import jax
import jax.numpy as jnp
from jax import lax
from jax.experimental import pallas as pl
from jax.experimental.pallas import tpu as pltpu

_LANES = 128
_CH = 128
_K = 16


def _bpr_kernel(idx_ref, q_ref, tab_hbm, pos_ref, neg_ref,
                tab, ltA, itA, ntA, ltB, itB, ntB, sems, *, d):
    t = pl.program_id(0)

    @pl.when(t == 0)
    def _load_tables():
        ct = pltpu.make_async_copy(tab_hbm, tab, sems.at[0])
        ct.start()
        ct.wait()

    step_b = _K * _CH
    lane = lax.broadcasted_iota(jnp.int32, (2, _LANES), 1)
    half = lax.broadcasted_iota(jnp.int32, (2, _LANES), 0)
    m2 = (lane // d == half).astype(jnp.float32)
    dn = (((1,), (1,)), ((), ()))

    def gather(c, lt_t, it_t, nt_t):
        base = c * _CH
        for j in range(_CH):
            o = base + j
            lt_t[pl.ds(j, 1), :] = tab[idx_ref[0, 0, o]]
            it_t[pl.ds(j, 1), :] = tab[idx_ref[0, 0, step_b + o]]
            nt_t[pl.ds(j, 1), :] = tab[idx_ref[0, 0, 2 * step_b + o]]

    def reduce(c, lt_t, it_t, nt_t):
        lt_ = lt_t[...]
        it_ = it_t[...]
        nt_ = nt_t[...]
        it_r = pltpu.roll(it_, d, axis=1)
        nt_r = pltpu.roll(nt_, d, axis=1)
        rp = lax.dot_general(m2, lt_ * it_, dn,
                             preferred_element_type=jnp.float32)
        rpr = lax.dot_general(m2, lt_ * it_r, dn,
                              preferred_element_type=jnp.float32)
        rn = lax.dot_general(m2, lt_ * nt_, dn,
                             preferred_element_type=jnp.float32)
        rnr = lax.dot_general(m2, lt_ * nt_r, dn,
                              preferred_element_type=jnp.float32)
        ql = q_ref[0, pl.ds(c, 1), :]
        qi = q_ref[0, pl.ds(_K + c, 1), :]
        qn = q_ref[0, pl.ds(2 * _K + c, 1), :]
        ql0 = ql == 0
        pos_ref[0, pl.ds(c, 1), :] = jnp.where(
            ql0,
            jnp.where(ql == qi, rp[0:1, :], rpr[0:1, :]),
            jnp.where(ql == qi, rp[1:2, :], rpr[1:2, :]))
        neg_ref[0, pl.ds(c, 1), :] = jnp.where(
            ql0,
            jnp.where(ql == qn, rn[0:1, :], rnr[0:1, :]),
            jnp.where(ql == qn, rn[1:2, :], rnr[1:2, :]))

    for c in range(_K):
        tiles = (ltA, itA, ntA) if c % 2 == 0 else (ltB, itB, ntB)
        gather(c, *tiles)
        reduce(c, *tiles)


@jax.jit
def kernel(list_table, item_table, list_indices, item_indices,
           item_neg_indices):
    B = list_indices.shape[0]
    D = list_table.shape[1]
    pk = _LANES // D

    step_b = _K * _CH
    Bp = -(-B // step_b) * step_b
    if Bp != B:
        pad = Bp - B
        list_indices = jnp.pad(list_indices, (0, pad))
        item_indices = jnp.pad(item_indices, (0, pad))
        item_neg_indices = jnp.pad(item_neg_indices, (0, pad))
    S = Bp // step_b
    nl = list_table.shape[0] // pk

    rows = jnp.concatenate(
        [(list_indices // pk).reshape(S, 1, step_b),
         (item_indices // pk + nl).reshape(S, 1, step_b),
         (item_neg_indices // pk + nl).reshape(S, 1, step_b)], axis=2)
    qall = jnp.concatenate(
        [(list_indices % pk).astype(jnp.int32).reshape(S, _K, _LANES),
         (item_indices % pk).astype(jnp.int32).reshape(S, _K, _LANES),
         (item_neg_indices % pk).astype(jnp.int32).reshape(S, _K, _LANES)],
        axis=1)

    tab3 = jnp.concatenate([list_table.reshape(-1, 1, _LANES),
                            item_table.reshape(-1, 1, _LANES)], axis=0)

    idx_spec = pl.BlockSpec((1, 1, 3 * step_b), lambda t: (t, 0, 0),
                            memory_space=pltpu.SMEM)
    q_spec = pl.BlockSpec((1, 3 * _K, _LANES), lambda t: (t, 0, 0))
    tab_spec = pl.BlockSpec(memory_space=pl.ANY)
    out_spec = pl.BlockSpec((1, _K, _LANES), lambda t: (t, 0, 0))

    pos3, neg3 = pl.pallas_call(
        partial(_bpr_kernel, d=D),
        grid=(S,),
        in_specs=[idx_spec, q_spec, tab_spec],
        out_specs=[out_spec, out_spec],
        out_shape=(
            jax.ShapeDtypeStruct((S, _K, _LANES), jnp.float32),
            jax.ShapeDtypeStruct((S, _K, _LANES), jnp.float32),
        ),
        scratch_shapes=[
            pltpu.VMEM(tab3.shape, jnp.float32),
        ] + [pltpu.VMEM((_CH, _LANES), jnp.float32)] * 6 + [
            pltpu.SemaphoreType.DMA((1,)),
        ],
        compiler_params=pltpu.CompilerParams(
            dimension_semantics=("arbitrary",),
            vmem_limit_bytes=56 * 1024 * 1024,
        ),
    )(rows, qall, tab3)
    return pos3.reshape(Bp)[:B], neg3.reshape(Bp)[:B]


# speedup vs baseline: 1.2957x; 1.2957x over previous
from functools import partial

import jax
import jax.numpy as jnp
from jax import lax
from jax.experimental import pallas as pl
from jax.experimental.pallas import tpu as pltpu

_LANES = 128
_CH = 128
_K = 16


def _bpr_kernel(lrow_ref, irow_ref, nrow_ref, q_ref, ltab_hbm, itab_hbm,
                pos_ref, neg_ref,
                ltab, itab, ltA, itA, ntA, ltB, itB, ntB, sems, *, d):
    t = pl.program_id(0)

    @pl.when(t == 0)
    def _load_tables():
        cl = pltpu.make_async_copy(ltab_hbm, ltab, sems.at[0])
        ci = pltpu.make_async_copy(itab_hbm, itab, sems.at[1])
        cl.start()
        ci.start()
        cl.wait()
        ci.wait()

    lane = lax.broadcasted_iota(jnp.int32, (2, _LANES), 1)
    half = lax.broadcasted_iota(jnp.int32, (2, _LANES), 0)
    m2 = (lane // d == half).astype(jnp.float32)
    dn = (((1,), (1,)), ((), ()))

    def gather(c, lt_t, it_t, nt_t):
        base = c * _CH
        for j in range(_CH):
            o = base + j
            lt_t[pl.ds(j, 1), :] = ltab[lrow_ref[0, 0, o]]
            it_t[pl.ds(j, 1), :] = itab[irow_ref[0, 0, o]]
            nt_t[pl.ds(j, 1), :] = itab[nrow_ref[0, 0, o]]

    def reduce(c, lt_t, it_t, nt_t):
        lt_ = lt_t[...]
        it_ = it_t[...]
        nt_ = nt_t[...]
        it_r = pltpu.roll(it_, d, axis=1)
        nt_r = pltpu.roll(nt_, d, axis=1)
        rp = lax.dot_general(m2, lt_ * it_, dn,
                             preferred_element_type=jnp.float32)
        rpr = lax.dot_general(m2, lt_ * it_r, dn,
                              preferred_element_type=jnp.float32)
        rn = lax.dot_general(m2, lt_ * nt_, dn,
                             preferred_element_type=jnp.float32)
        rnr = lax.dot_general(m2, lt_ * nt_r, dn,
                              preferred_element_type=jnp.float32)
        ql = q_ref[0, pl.ds(c, 1), :]
        qi = q_ref[0, pl.ds(_K + c, 1), :]
        qn = q_ref[0, pl.ds(2 * _K + c, 1), :]
        ql0 = ql == 0
        pos_ref[0, pl.ds(c, 1), :] = jnp.where(
            ql0,
            jnp.where(ql == qi, rp[0:1, :], rpr[0:1, :]),
            jnp.where(ql == qi, rp[1:2, :], rpr[1:2, :]))
        neg_ref[0, pl.ds(c, 1), :] = jnp.where(
            ql0,
            jnp.where(ql == qn, rn[0:1, :], rnr[0:1, :]),
            jnp.where(ql == qn, rn[1:2, :], rnr[1:2, :]))

    for c in range(_K):
        tiles = (ltA, itA, ntA) if c % 2 == 0 else (ltB, itB, ntB)
        gather(c, *tiles)
        reduce(c, *tiles)


@jax.jit
def kernel(list_table, item_table, list_indices, item_indices,
           item_neg_indices):
    B = list_indices.shape[0]
    D = list_table.shape[1]
    pk = _LANES // D

    step_b = _K * _CH
    Bp = -(-B // step_b) * step_b
    if Bp != B:
        pad = Bp - B
        list_indices = jnp.pad(list_indices, (0, pad))
        item_indices = jnp.pad(item_indices, (0, pad))
        item_neg_indices = jnp.pad(item_neg_indices, (0, pad))
    S = Bp // step_b

    lrow = (list_indices // pk).reshape(S, 1, step_b)
    irow = (item_indices // pk).reshape(S, 1, step_b)
    nrow = (item_neg_indices // pk).reshape(S, 1, step_b)
    qall = jnp.concatenate(
        [(list_indices % pk).astype(jnp.int32).reshape(S, _K, _LANES),
         (item_indices % pk).astype(jnp.int32).reshape(S, _K, _LANES),
         (item_neg_indices % pk).astype(jnp.int32).reshape(S, _K, _LANES)],
        axis=1)

    ltab3 = list_table.reshape(-1, 1, _LANES)
    itab3 = item_table.reshape(-1, 1, _LANES)

    idx_spec = pl.BlockSpec((1, 1, step_b), lambda t: (t, 0, 0),
                            memory_space=pltpu.SMEM)
    q_spec = pl.BlockSpec((1, 3 * _K, _LANES), lambda t: (t, 0, 0))
    tab_spec = pl.BlockSpec(memory_space=pl.ANY)
    out_spec = pl.BlockSpec((1, _K, _LANES), lambda t: (t, 0, 0))

    pos3, neg3 = pl.pallas_call(
        partial(_bpr_kernel, d=D),
        grid=(S,),
        in_specs=[idx_spec, idx_spec, idx_spec, q_spec, tab_spec, tab_spec],
        out_specs=[out_spec, out_spec],
        out_shape=(
            jax.ShapeDtypeStruct((S, _K, _LANES), jnp.float32),
            jax.ShapeDtypeStruct((S, _K, _LANES), jnp.float32),
        ),
        scratch_shapes=[
            pltpu.VMEM(ltab3.shape, jnp.float32),
            pltpu.VMEM(itab3.shape, jnp.float32),
        ] + [pltpu.VMEM((_CH, _LANES), jnp.float32)] * 6 + [
            pltpu.SemaphoreType.DMA((2,)),
        ],
        compiler_params=pltpu.CompilerParams(
            dimension_semantics=("arbitrary",),
            vmem_limit_bytes=56 * 1024 * 1024,
        ),
    )(lrow, irow, nrow, qall, ltab3, itab3)
    return pos3.reshape(Bp)[:B], neg3.reshape(Bp)[:B]


# speedup vs baseline: 1.3172x; 1.0167x over previous
from functools import partial

import jax
import jax.numpy as jnp
from jax import lax
from jax.experimental import pallas as pl
from jax.experimental.pallas import tpu as pltpu

_LANES = 128
_CH = 128
_K = 32


def _bpr_kernel(lrow_ref, irow_ref, nrow_ref, q_ref, ltab_hbm, itab_hbm,
                pos_ref, neg_ref,
                ltab, itab, ltA, itA, ntA, ltB, itB, ntB, sems, *, d):
    t = pl.program_id(0)

    @pl.when(t == 0)
    def _load_tables():
        cl = pltpu.make_async_copy(ltab_hbm, ltab, sems.at[0])
        ci = pltpu.make_async_copy(itab_hbm, itab, sems.at[1])
        cl.start()
        ci.start()
        cl.wait()
        ci.wait()

    lane = lax.broadcasted_iota(jnp.int32, (2, _LANES), 1)
    half = lax.broadcasted_iota(jnp.int32, (2, _LANES), 0)
    m2 = (lane // d == half).astype(jnp.float32)
    dn = (((1,), (1,)), ((), ()))

    def gather(c, lt_t, it_t, nt_t):
        base = c * _CH
        for j in range(_CH):
            o = base + j
            lt_t[pl.ds(j, 1), :] = ltab[lrow_ref[0, 0, o]]
            it_t[pl.ds(j, 1), :] = itab[irow_ref[0, 0, o]]
            nt_t[pl.ds(j, 1), :] = itab[nrow_ref[0, 0, o]]

    def reduce(c, lt_t, it_t, nt_t):
        lt_ = lt_t[...]
        it_ = it_t[...]
        nt_ = nt_t[...]
        it_r = pltpu.roll(it_, d, axis=1)
        nt_r = pltpu.roll(nt_, d, axis=1)
        rp = lax.dot_general(m2, lt_ * it_, dn,
                             preferred_element_type=jnp.float32)
        rpr = lax.dot_general(m2, lt_ * it_r, dn,
                              preferred_element_type=jnp.float32)
        rn = lax.dot_general(m2, lt_ * nt_, dn,
                             preferred_element_type=jnp.float32)
        rnr = lax.dot_general(m2, lt_ * nt_r, dn,
                              preferred_element_type=jnp.float32)
        ql = q_ref[0, pl.ds(c, 1), :]
        qi = q_ref[0, pl.ds(_K + c, 1), :]
        qn = q_ref[0, pl.ds(2 * _K + c, 1), :]
        ql0 = ql == 0
        pos_ref[0, pl.ds(c, 1), :] = jnp.where(
            ql0,
            jnp.where(ql == qi, rp[0:1, :], rpr[0:1, :]),
            jnp.where(ql == qi, rp[1:2, :], rpr[1:2, :]))
        neg_ref[0, pl.ds(c, 1), :] = jnp.where(
            ql0,
            jnp.where(ql == qn, rn[0:1, :], rnr[0:1, :]),
            jnp.where(ql == qn, rn[1:2, :], rnr[1:2, :]))

    for c in range(_K):
        tiles = (ltA, itA, ntA) if c % 2 == 0 else (ltB, itB, ntB)
        gather(c, *tiles)
        reduce(c, *tiles)


@jax.jit
def kernel(list_table, item_table, list_indices, item_indices,
           item_neg_indices):
    B = list_indices.shape[0]
    D = list_table.shape[1]
    pk = _LANES // D

    step_b = _K * _CH
    Bp = -(-B // step_b) * step_b
    if Bp != B:
        pad = Bp - B
        list_indices = jnp.pad(list_indices, (0, pad))
        item_indices = jnp.pad(item_indices, (0, pad))
        item_neg_indices = jnp.pad(item_neg_indices, (0, pad))
    S = Bp // step_b

    lrow = (list_indices // pk).reshape(S, 1, step_b)
    irow = (item_indices // pk).reshape(S, 1, step_b)
    nrow = (item_neg_indices // pk).reshape(S, 1, step_b)
    qall = jnp.concatenate(
        [(list_indices % pk).astype(jnp.int32).reshape(S, _K, _LANES),
         (item_indices % pk).astype(jnp.int32).reshape(S, _K, _LANES),
         (item_neg_indices % pk).astype(jnp.int32).reshape(S, _K, _LANES)],
        axis=1)

    ltab3 = list_table.reshape(-1, 1, _LANES)
    itab3 = item_table.reshape(-1, 1, _LANES)

    idx_spec = pl.BlockSpec((1, 1, step_b), lambda t: (t, 0, 0),
                            memory_space=pltpu.SMEM)
    q_spec = pl.BlockSpec((1, 3 * _K, _LANES), lambda t: (t, 0, 0))
    tab_spec = pl.BlockSpec(memory_space=pl.ANY)
    out_spec = pl.BlockSpec((1, _K, _LANES), lambda t: (t, 0, 0))

    pos3, neg3 = pl.pallas_call(
        partial(_bpr_kernel, d=D),
        grid=(S,),
        in_specs=[idx_spec, idx_spec, idx_spec, q_spec, tab_spec, tab_spec],
        out_specs=[out_spec, out_spec],
        out_shape=(
            jax.ShapeDtypeStruct((S, _K, _LANES), jnp.float32),
            jax.ShapeDtypeStruct((S, _K, _LANES), jnp.float32),
        ),
        scratch_shapes=[
            pltpu.VMEM(ltab3.shape, jnp.float32),
            pltpu.VMEM(itab3.shape, jnp.float32),
        ] + [pltpu.VMEM((_CH, _LANES), jnp.float32)] * 6 + [
            pltpu.SemaphoreType.DMA((2,)),
        ],
        compiler_params=pltpu.CompilerParams(
            dimension_semantics=("arbitrary",),
            vmem_limit_bytes=56 * 1024 * 1024,
        ),
    )(lrow, irow, nrow, qall, ltab3, itab3)
    return pos3.reshape(Bp)[:B], neg3.reshape(Bp)[:B]


# speedup vs baseline: 1.3173x; 1.0001x over previous
from functools import partial

import jax
import jax.numpy as jnp
from jax import lax
from jax.experimental import pallas as pl
from jax.experimental.pallas import tpu as pltpu

_LANES = 128
_CH = 128
_K = 32


def _bpr_kernel(lrow_ref, irow_ref, nrow_ref, q_ref, ltab_hbm, itab_hbm,
                pos_ref, neg_ref,
                ltab, itab, ltA, itA, ntA, ltB, itB, ntB, sems, *, d):
    t = pl.program_id(0)

    @pl.when(t == 0)
    def _load_tables():
        nl2 = ltab.shape[0] // 2
        ni2 = itab.shape[0] // 2
        cps = [
            pltpu.make_async_copy(ltab_hbm.at[pl.ds(0, nl2)],
                                  ltab.at[pl.ds(0, nl2)], sems.at[0]),
            pltpu.make_async_copy(ltab_hbm.at[pl.ds(nl2, nl2)],
                                  ltab.at[pl.ds(nl2, nl2)], sems.at[1]),
            pltpu.make_async_copy(itab_hbm.at[pl.ds(0, ni2)],
                                  itab.at[pl.ds(0, ni2)], sems.at[2]),
            pltpu.make_async_copy(itab_hbm.at[pl.ds(ni2, ni2)],
                                  itab.at[pl.ds(ni2, ni2)], sems.at[3]),
        ]
        for cp in cps:
            cp.start()
        for cp in cps:
            cp.wait()

    lane = lax.broadcasted_iota(jnp.int32, (2, _LANES), 1)
    half = lax.broadcasted_iota(jnp.int32, (2, _LANES), 0)
    m2 = (lane // d == half).astype(jnp.float32)
    dn = (((1,), (1,)), ((), ()))

    def gather(c, lt_t, it_t, nt_t):
        base = c * _CH
        for j in range(_CH):
            o = base + j
            lt_t[pl.ds(j, 1), :] = ltab[lrow_ref[0, 0, o]]
            it_t[pl.ds(j, 1), :] = itab[irow_ref[0, 0, o]]
            nt_t[pl.ds(j, 1), :] = itab[nrow_ref[0, 0, o]]

    def reduce(c, lt_t, it_t, nt_t):
        lt_ = lt_t[...]
        it_ = it_t[...]
        nt_ = nt_t[...]
        it_r = pltpu.roll(it_, d, axis=1)
        nt_r = pltpu.roll(nt_, d, axis=1)
        rp = lax.dot_general(m2, lt_ * it_, dn,
                             preferred_element_type=jnp.float32)
        rpr = lax.dot_general(m2, lt_ * it_r, dn,
                              preferred_element_type=jnp.float32)
        rn = lax.dot_general(m2, lt_ * nt_, dn,
                             preferred_element_type=jnp.float32)
        rnr = lax.dot_general(m2, lt_ * nt_r, dn,
                              preferred_element_type=jnp.float32)
        ql = q_ref[0, pl.ds(c, 1), :]
        qi = q_ref[0, pl.ds(_K + c, 1), :]
        qn = q_ref[0, pl.ds(2 * _K + c, 1), :]
        ql0 = ql == 0
        pos_ref[0, pl.ds(c, 1), :] = jnp.where(
            ql0,
            jnp.where(ql == qi, rp[0:1, :], rpr[0:1, :]),
            jnp.where(ql == qi, rp[1:2, :], rpr[1:2, :]))
        neg_ref[0, pl.ds(c, 1), :] = jnp.where(
            ql0,
            jnp.where(ql == qn, rn[0:1, :], rnr[0:1, :]),
            jnp.where(ql == qn, rn[1:2, :], rnr[1:2, :]))

    for c in range(_K):
        tiles = (ltA, itA, ntA) if c % 2 == 0 else (ltB, itB, ntB)
        gather(c, *tiles)
        reduce(c, *tiles)


@jax.jit
def kernel(list_table, item_table, list_indices, item_indices,
           item_neg_indices):
    B = list_indices.shape[0]
    D = list_table.shape[1]
    pk = _LANES // D

    step_b = _K * _CH
    Bp = -(-B // step_b) * step_b
    if Bp != B:
        pad = Bp - B
        list_indices = jnp.pad(list_indices, (0, pad))
        item_indices = jnp.pad(item_indices, (0, pad))
        item_neg_indices = jnp.pad(item_neg_indices, (0, pad))
    S = Bp // step_b

    lrow = (list_indices // pk).reshape(S, 1, step_b)
    irow = (item_indices // pk).reshape(S, 1, step_b)
    nrow = (item_neg_indices // pk).reshape(S, 1, step_b)
    qall = jnp.concatenate(
        [(list_indices % pk).astype(jnp.int32).reshape(S, _K, _LANES),
         (item_indices % pk).astype(jnp.int32).reshape(S, _K, _LANES),
         (item_neg_indices % pk).astype(jnp.int32).reshape(S, _K, _LANES)],
        axis=1)

    ltab3 = list_table.reshape(-1, 1, _LANES)
    itab3 = item_table.reshape(-1, 1, _LANES)

    idx_spec = pl.BlockSpec((1, 1, step_b), lambda t: (t, 0, 0),
                            memory_space=pltpu.SMEM)
    q_spec = pl.BlockSpec((1, 3 * _K, _LANES), lambda t: (t, 0, 0))
    tab_spec = pl.BlockSpec(memory_space=pl.ANY)
    out_spec = pl.BlockSpec((1, _K, _LANES), lambda t: (t, 0, 0))

    pos3, neg3 = pl.pallas_call(
        partial(_bpr_kernel, d=D),
        grid=(S,),
        in_specs=[idx_spec, idx_spec, idx_spec, q_spec, tab_spec, tab_spec],
        out_specs=[out_spec, out_spec],
        out_shape=(
            jax.ShapeDtypeStruct((S, _K, _LANES), jnp.float32),
            jax.ShapeDtypeStruct((S, _K, _LANES), jnp.float32),
        ),
        scratch_shapes=[
            pltpu.VMEM(ltab3.shape, jnp.float32),
            pltpu.VMEM(itab3.shape, jnp.float32),
        ] + [pltpu.VMEM((_CH, _LANES), jnp.float32)] * 6 + [
            pltpu.SemaphoreType.DMA((4,)),
        ],
        compiler_params=pltpu.CompilerParams(
            dimension_semantics=("arbitrary",),
            vmem_limit_bytes=56 * 1024 * 1024,
        ),
    )(lrow, irow, nrow, qall, ltab3, itab3)
    return pos3.reshape(Bp)[:B], neg3.reshape(Bp)[:B]
